# baseline (speedup 1.0000x reference)
import jax
import jax.numpy as jnp
from jax.experimental import pallas as pl

_B = 1024
_F = 100
_D = 16
_V = 100
_TP = 128
_TW = 64


def _prep_kernel(x_ref, xt_ref, wcol_ref, et_ref, et2_ref, hc_ref, hl_ref):
    xb = x_ref[...]
    iota = jax.lax.broadcasted_iota(jnp.int32, (1, 1, _V), 2)
    eq = (xb[:, :, None] == iota).astype(jnp.int32)
    cmat = jnp.sum(eq, axis=1).astype(jnp.float32)
    tr = _TP * _D
    cexp = jnp.broadcast_to(cmat[:, None, :], (_TP, _D, _V)).reshape(tr, _V)
    eg = jnp.broadcast_to(et_ref[...][None, :, :], (_TP, _D, _V)).reshape(tr, _V)
    eg2 = jnp.broadcast_to(et2_ref[...][None, :, :], (_TP, _D, _V)).reshape(tr, _V)
    se = jnp.sum(cexp * eg, axis=1, keepdims=True)
    se2 = jnp.sum(cexp * eg2, axis=1, keepdims=True)
    hc_ref[...] = 0.25 * (se * se) - 0.25 * se2
    lin = jnp.sum(wcol_ref[...] * xt_ref[...], axis=0, keepdims=True)
    hl_ref[...] = 0.5 * lin


def _writer_kernel(hc_ref, hl_ref, out_ref):
    t = hc_ref[...] + hl_ref[...]
    out_ref[...] = 0.5 * jnp.tanh(t) + 0.5


def kernel(x, emb_table, linear_weights):
    xt = x.astype(jnp.float32).T
    wcol = linear_weights.reshape(_F, 1)
    et = emb_table.T
    et2 = et * et

    half_cross, half_lin = pl.pallas_call(
        _prep_kernel,
        grid=(_B // _TP,),
        in_specs=[
            pl.BlockSpec((_TP, _F), lambda i: (i, 0)),
            pl.BlockSpec((_F, _TP), lambda i: (0, i)),
            pl.BlockSpec((_F, 1), lambda i: (0, 0)),
            pl.BlockSpec((_D, _V), lambda i: (0, 0)),
            pl.BlockSpec((_D, _V), lambda i: (0, 0)),
        ],
        out_specs=[
            pl.BlockSpec((_TP * _D, 1), lambda i: (i, 0)),
            pl.BlockSpec((1, _TP), lambda i: (0, i)),
        ],
        out_shape=[
            jax.ShapeDtypeStruct((_B * _D, 1), jnp.float32),
            jax.ShapeDtypeStruct((1, _B), jnp.float32),
        ],
    )(x, xt, wcol, et, et2)

    out2 = pl.pallas_call(
        _writer_kernel,
        grid=(_B // _TW,),
        in_specs=[
            pl.BlockSpec((_TW * _D, 1), lambda i: (i, 0)),
            pl.BlockSpec((1, _B), lambda i: (0, 0)),
        ],
        out_specs=pl.BlockSpec((_TW * _D, _B), lambda i: (i, 0)),
        out_shape=jax.ShapeDtypeStruct((_B * _D, _B), jnp.float32),
    )(half_cross, half_lin)

    return out2.reshape(_B, _D, _B).transpose(0, 2, 1)


# speedup vs baseline: 1.2426x; 1.2426x over previous
import jax
import jax.numpy as jnp
from jax.experimental import pallas as pl
from jax.experimental.pallas import tpu as pltpu

_B = 1024
_F = 100
_D = 16
_V = 100
_TI = 64
_TR = _TI * _D


def _fm_kernel(x_ref, xt_ref, wcol_ref, et_ref, et2_ref, out_ref):
    xb = x_ref[...]
    iota = jax.lax.broadcasted_iota(jnp.int32, (1, 1, _V), 2)
    eq = (xb[:, :, None] == iota).astype(jnp.int32)
    cmat = jnp.sum(eq, axis=1).astype(jnp.float32)
    cexp = jnp.broadcast_to(cmat[:, None, :], (_TI, _D, _V)).reshape(_TR, _V)
    eg = jnp.broadcast_to(et_ref[...][None, :, :], (_TI, _D, _V)).reshape(_TR, _V)
    eg2 = jnp.broadcast_to(et2_ref[...][None, :, :], (_TI, _D, _V)).reshape(_TR, _V)
    se = jnp.sum(cexp * eg, axis=1, keepdims=True)
    se2 = jnp.sum(cexp * eg2, axis=1, keepdims=True)
    half_cross = 0.25 * (se * se) - 0.25 * se2
    lin_row = jnp.sum(wcol_ref[...] * xt_ref[...], axis=0, keepdims=True)
    half_lin = 0.5 * lin_row
    t = half_cross + half_lin
    out_ref[...] = 0.5 * jnp.tanh(t) + 0.5


def kernel(x, emb_table, linear_weights):
    n_i = _B // _TI
    xt = x.astype(jnp.float32).T
    wcol = linear_weights.reshape(_F, 1)
    et = emb_table.T
    et2 = et * et

    out2 = pl.pallas_call(
        _fm_kernel,
        grid=(n_i,),
        in_specs=[
            pl.BlockSpec((_TI, _F), lambda i: (i, 0)),
            pl.BlockSpec((_F, _B), lambda i: (0, 0)),
            pl.BlockSpec((_F, 1), lambda i: (0, 0)),
            pl.BlockSpec((_D, _V), lambda i: (0, 0)),
            pl.BlockSpec((_D, _V), lambda i: (0, 0)),
        ],
        out_specs=pl.BlockSpec((_TR, _B), lambda i: (i, 0)),
        out_shape=jax.ShapeDtypeStruct((_B * _D, _B), jnp.float32),
        compiler_params=pltpu.CompilerParams(
            dimension_semantics=("parallel",),
        ),
    )(x, xt, wcol, et, et2)

    return out2.reshape(_B, _D, _B).transpose(0, 2, 1)
